# baseline (speedup 1.0000x reference)
import functools

import jax
import jax.numpy as jnp
from jax import lax
from jax.experimental import pallas as pl
from jax.experimental.pallas import tpu as pltpu
from jax.experimental.pallas import tpu_sc as plsc

TIME = 288
FEATURES = 64
B, T, N, C = 64, 12, 4096, 3

NUM_CORES = 2
NUM_SUBCORES = 16
FHALF = FEATURES // NUM_CORES
B_PER_W = B // NUM_SUBCORES
LANES = 16
NCHUNKS = N // LANES
FBLK = 4
NBLKS = FHALF // FBLK
NCOMB = TIME * 7


def _body(xs_hbm, td_hbm, tw_hbm, out_hbm,
          td_v, tw_v, comb_v, xd_v, xw_v, idxc_v, row_v, sem0, sem1):
    sems = (sem0, sem1)
    cid = lax.axis_index("c")
    sid = lax.axis_index("s")
    fbase = cid * FHALF
    iota = lax.iota(jnp.int32, LANES)

    pltpu.sync_copy(td_hbm, td_v)
    pltpu.sync_copy(tw_hbm, tw_v)

    for fl in range(FHALF):
        f = fbase + fl

        def wbody(w, _, fl=fl, f=f):
            wv = plsc.load_gather(
                tw_v, [jnp.zeros((LANES,), jnp.int32) + (f * 7 + w)])

            @plsc.parallel_loop(0, TIME // LANES, unroll=2)
            def bld(v, fl=fl, f=f, w=w, wv=wv):
                vals = td_v[pl.ds(f * TIME + v * LANES, LANES)] + wv
                didx = (v * LANES + iota) * 7 + (fl * NCOMB + w)
                plsc.store_scatter(comb_v, [didx], vals)

            return 0

        lax.fori_loop(0, 7, wbody, 0)

    for b_local in range(B_PER_W):
        b = sid * B_PER_W + b_local

        pltpu.sync_copy(xs_hbm.at[b, 0], xd_v)
        pltpu.sync_copy(xs_hbm.at[b, 1], xw_v)

        @plsc.parallel_loop(0, NCHUNKS, unroll=4)
        def idx_body(i):
            sl = pl.ds(i * LANES, LANES)
            dayv = xd_v[sl]
            weekv = xw_v[sl]
            d = jnp.clip((dayv * float(TIME)).astype(jnp.int32), 0, TIME - 1)
            w = jnp.clip(weekv.astype(jnp.int32), 0, 6)
            idxc_v[sl] = d * 7 + w

        pending = {0: None, 1: None}
        for fblk in range(NBLKS):
            ph = fblk % 2
            if pending[ph] is not None:
                pending[ph].wait()

            @plsc.parallel_loop(0, NCHUNKS, unroll=4)
            def gather_body(i, ph=ph, fblk=fblk):
                sl = pl.ds(i * LANES, LANES)
                cvec = idxc_v[sl]
                for j in range(FBLK):
                    fl = fblk * FBLK + j
                    val = plsc.load_gather(comb_v, [cvec + fl * NCOMB])
                    row_v[ph, pl.ds(j * N + i * LANES, LANES)] = val

            pending[ph] = pltpu.async_copy(
                row_v.at[ph],
                out_hbm.at[pl.ds((b * FEATURES + fbase + fblk * FBLK) * N,
                                 FBLK * N)],
                sems[ph])

        for ph in (0, 1):
            if pending[ph] is not None:
                pending[ph].wait()


def kernel(x, time_day, time_week):
    xs = jnp.transpose(x[:, -1, :, 1:3], (0, 2, 1))
    td = time_day.T.reshape(-1)
    tw = time_week.T.reshape(-1)

    mesh = plsc.VectorSubcoreMesh(
        core_axis_name="c", subcore_axis_name="s",
        num_cores=NUM_CORES, num_subcores=NUM_SUBCORES)
    run = functools.partial(
        pl.kernel,
        out_type=jax.ShapeDtypeStruct((B * FEATURES * N,), jnp.float32),
        mesh=mesh,
        compiler_params=pltpu.CompilerParams(needs_layout_passes=False),
        scratch_types=[
            pltpu.VMEM((FEATURES * TIME,), jnp.float32),
            pltpu.VMEM((FEATURES * 7,), jnp.float32),
            pltpu.VMEM((FHALF * NCOMB,), jnp.float32),
            pltpu.VMEM((N,), jnp.float32),
            pltpu.VMEM((N,), jnp.float32),
            pltpu.VMEM((N,), jnp.int32),
            pltpu.VMEM((2, FBLK * N), jnp.float32),
            pltpu.SemaphoreType.DMA,
            pltpu.SemaphoreType.DMA,
        ],
    )(_body)
    out = run(xs, td, tw)
    return out.reshape(B, FEATURES, N, 1)


# speedup vs baseline: 1.0222x; 1.0222x over previous
import functools

import jax
import jax.numpy as jnp
from jax import lax
from jax.experimental import pallas as pl
from jax.experimental.pallas import tpu as pltpu
from jax.experimental.pallas import tpu_sc as plsc

TIME = 288
FEATURES = 64
B, T, N, C = 64, 12, 4096, 3

NUM_CORES = 2
NUM_SUBCORES = 16
NUM_WORKERS = NUM_CORES * NUM_SUBCORES
B_PER_W = B // NUM_WORKERS
LANES = 16
NCHUNKS = N // LANES
FBLK = 8
NBLKS = FEATURES // FBLK
TW_PAD = 7 * FEATURES + LANES


def _body(xf_hbm, td_hbm, tw_hbm, out_hbm,
          td_v, tw_v, xdw_v, idxc_v, row_v, semx0, semx1, sem0, sem1):
    sems = (sem0, sem1)
    wid = lax.axis_index("s") * NUM_CORES + lax.axis_index("c")
    iota = lax.iota(jnp.int32, LANES)

    xdmas = [
        pltpu.async_copy(xf_hbm.at[wid * B_PER_W + bl],
                         xdw_v.at[pl.ds(bl * 2 * N, 2 * N)], semx)
        for bl, semx in ((0, semx0), (1, semx1))
    ]

    pltpu.sync_copy(td_hbm, td_v)
    pltpu.sync_copy(tw_hbm, tw_v)

    for b_local in range(B_PER_W):
        b = wid * B_PER_W + b_local
        xdmas[b_local].wait()

        @plsc.parallel_loop(0, NCHUNKS, unroll=4)
        def idx_body(i, b_local=b_local):
            base = b_local * 2 * N + i * 2 * LANES
            dayv = plsc.load_gather(xdw_v, [base + iota * 2])
            weekv = plsc.load_gather(xdw_v, [base + iota * 2 + 1])
            d = jnp.clip((dayv * float(TIME)).astype(jnp.int32), 0, TIME - 1)
            w = jnp.clip(weekv.astype(jnp.int32), 0, 6)
            idxc_v[pl.ds(i * LANES, LANES)] = d * 8 + w

        pending = {0: None, 1: None}
        for fblk in range(NBLKS):
            ph = fblk % 2
            if pending[ph] is not None:
                pending[ph].wait()

            twrows = [tw_v[pl.ds((fblk * FBLK + j) * 7, LANES)]
                      for j in range(FBLK)]

            @plsc.parallel_loop(0, NCHUNKS, unroll=2)
            def gather_body(i, ph=ph, fblk=fblk, twrows=twrows):
                sl = pl.ds(i * LANES, LANES)
                cvec = idxc_v[sl]
                dvec = cvec >> 3
                wvec = cvec & 7
                for j in range(FBLK):
                    f = fblk * FBLK + j
                    dayv = plsc.load_gather(td_v, [dvec + f * TIME])
                    weekv = lax.gather(
                        twrows[j], wvec[:, None],
                        dimension_numbers=lax.GatherDimensionNumbers(
                            offset_dims=(), collapsed_slice_dims=(0,),
                            start_index_map=(0,)),
                        slice_sizes=(1,),
                        mode=lax.GatherScatterMode.PROMISE_IN_BOUNDS)
                    row_v[ph, pl.ds(j * N + i * LANES, LANES)] = dayv + weekv

            pending[ph] = pltpu.async_copy(
                row_v.at[ph],
                out_hbm.at[pl.ds((b * FEATURES + fblk * FBLK) * N,
                                 FBLK * N)],
                sems[ph])

        for ph in (0, 1):
            if pending[ph] is not None:
                pending[ph].wait()


def kernel(x, time_day, time_week):
    xf = x[:, -1, :, 1:3].reshape(B, 2 * N)
    td = time_day.T.reshape(-1)
    tw = jnp.pad(time_week.T.reshape(-1), (0, LANES))

    mesh = plsc.VectorSubcoreMesh(
        core_axis_name="c", subcore_axis_name="s",
        num_cores=NUM_CORES, num_subcores=NUM_SUBCORES)
    run = functools.partial(
        pl.kernel,
        out_type=jax.ShapeDtypeStruct((B * FEATURES * N,), jnp.float32),
        mesh=mesh,
        compiler_params=pltpu.CompilerParams(needs_layout_passes=False),
        scratch_types=[
            pltpu.VMEM((FEATURES * TIME,), jnp.float32),
            pltpu.VMEM((TW_PAD,), jnp.float32),
            pltpu.VMEM((B_PER_W * 2 * N,), jnp.float32),
            pltpu.VMEM((N,), jnp.int32),
            pltpu.VMEM((2, FBLK * N), jnp.float32),
            pltpu.SemaphoreType.DMA,
            pltpu.SemaphoreType.DMA,
            pltpu.SemaphoreType.DMA,
            pltpu.SemaphoreType.DMA,
        ],
    )(_body)
    out = run(xf, td, tw)
    return out.reshape(B, FEATURES, N, 1)


# speedup vs baseline: 1.1197x; 1.0953x over previous
import functools

import jax
import jax.numpy as jnp
from jax import lax
from jax.experimental import pallas as pl
from jax.experimental.pallas import tpu as pltpu
from jax.experimental.pallas import tpu_sc as plsc

TIME = 288
FEATURES = 64
B, T, N, C = 64, 12, 4096, 3

NUM_CORES = 2
NUM_SUBCORES = 16
NUM_WORKERS = NUM_CORES * NUM_SUBCORES
B_PER_W = B // NUM_WORKERS
LANES = 16
NCHUNKS = N // LANES
FBLK = 8
NBLKS = FEATURES // FBLK
TW_PAD = 7 * FEATURES + LANES


def _body(xd_hbm, xw_hbm, td_hbm, tw_hbm, out_hbm,
          td_v, tw_v, xdw_v, idxc_v, row_v,
          semx0, semx1, semx2, semx3, sem0, sem1):
    sems = (sem0, sem1)
    xsems = (semx0, semx1, semx2, semx3)
    wid = lax.axis_index("s") * NUM_CORES + lax.axis_index("c")

    xdmas = [
        pltpu.async_copy(src.at[wid * B_PER_W + bl],
                         xdw_v.at[pl.ds((bl * 2 + k) * N, N)],
                         xsems[bl * 2 + k])
        for bl in range(B_PER_W)
        for k, src in ((0, xd_hbm), (1, xw_hbm))
    ]

    pltpu.sync_copy(td_hbm, td_v)
    pltpu.sync_copy(tw_hbm, tw_v)

    for b_local in range(B_PER_W):
        b = wid * B_PER_W + b_local
        xdmas[b_local * 2].wait()
        xdmas[b_local * 2 + 1].wait()

        @plsc.parallel_loop(0, NCHUNKS, unroll=4)
        def idx_body(i, b_local=b_local):
            base = b_local * 2 * N + i * LANES
            dayv = xdw_v[pl.ds(base, LANES)]
            weekv = xdw_v[pl.ds(base + N, LANES)]
            d = jnp.clip((dayv * float(TIME)).astype(jnp.int32), 0, TIME - 1)
            w = jnp.clip(weekv.astype(jnp.int32), 0, 6)
            idxc_v[pl.ds(i * LANES, LANES)] = d * 8 + w

        pending = {0: None, 1: None}
        for fblk in range(NBLKS):
            ph = fblk % 2
            if pending[ph] is not None:
                pending[ph].wait()

            twrows = [tw_v[pl.ds((fblk * FBLK + j) * 7, LANES)]
                      for j in range(FBLK)]

            @plsc.parallel_loop(0, NCHUNKS, unroll=2)
            def gather_body(i, ph=ph, fblk=fblk, twrows=twrows):
                sl = pl.ds(i * LANES, LANES)
                cvec = idxc_v[sl]
                dvec = cvec >> 3
                wvec = cvec & 7
                for j in range(FBLK):
                    f = fblk * FBLK + j
                    dayv = plsc.load_gather(td_v, [dvec + f * TIME])
                    weekv = lax.gather(
                        twrows[j], wvec[:, None],
                        dimension_numbers=lax.GatherDimensionNumbers(
                            offset_dims=(), collapsed_slice_dims=(0,),
                            start_index_map=(0,)),
                        slice_sizes=(1,),
                        mode=lax.GatherScatterMode.PROMISE_IN_BOUNDS)
                    row_v[ph, pl.ds(j * N + i * LANES, LANES)] = dayv + weekv

            pending[ph] = pltpu.async_copy(
                row_v.at[ph],
                out_hbm.at[pl.ds((b * FEATURES + fblk * FBLK) * N,
                                 FBLK * N)],
                sems[ph])

        for ph in (0, 1):
            if pending[ph] is not None:
                pending[ph].wait()


def kernel(x, time_day, time_week):
    xd = x[:, -1, :, 1]
    xw = x[:, -1, :, 2]
    td = time_day.T.reshape(-1)
    tw = jnp.pad(time_week.T.reshape(-1), (0, LANES))

    mesh = plsc.VectorSubcoreMesh(
        core_axis_name="c", subcore_axis_name="s",
        num_cores=NUM_CORES, num_subcores=NUM_SUBCORES)
    run = functools.partial(
        pl.kernel,
        out_type=jax.ShapeDtypeStruct((B * FEATURES * N,), jnp.float32),
        mesh=mesh,
        compiler_params=pltpu.CompilerParams(needs_layout_passes=False),
        scratch_types=[
            pltpu.VMEM((FEATURES * TIME,), jnp.float32),
            pltpu.VMEM((TW_PAD,), jnp.float32),
            pltpu.VMEM((B_PER_W * 2 * N,), jnp.float32),
            pltpu.VMEM((N,), jnp.int32),
            pltpu.VMEM((2, FBLK * N), jnp.float32),
            pltpu.SemaphoreType.DMA,
            pltpu.SemaphoreType.DMA,
            pltpu.SemaphoreType.DMA,
            pltpu.SemaphoreType.DMA,
            pltpu.SemaphoreType.DMA,
            pltpu.SemaphoreType.DMA,
        ],
    )(_body)
    out = run(xd, xw, td, tw)
    return out.reshape(B, FEATURES, N, 1)
